# baseline (speedup 1.0000x reference)
import jax
import jax.numpy as jnp
from jax import lax
from jax.experimental import pallas as pl
from jax.experimental.pallas import tpu as pltpu
from jax.experimental.pallas import tpu_sc as plsc

T, H, E, K, FF = 2048, 1024, 8, 2, 2816
M = 256
PADDED = T * K + E * M
NB = PADDED // M
FF_BLK = 256
NJ = FF // FF_BLK
EPAD = 128
HS = H // 256

NC, NS = 2, 16
NW = NC * NS
G_ROWS = T * K // NW
C_CH = 32
C_NCH = (T // NW) // C_CH


def _router_body(x_ref, gw_ref, e0_ref, e1_ref, w0_ref):
    logits = lax.dot_general(x_ref[...], gw_ref[...],
                             (((1,), (1,)), ((), ())),
                             preferred_element_type=jnp.float32)
    rows = logits.shape[0]
    iota = lax.broadcasted_iota(jnp.int32, (rows, EPAD), 1)
    masked = jnp.where(iota < E, logits, -1e30)
    m0 = jnp.max(masked, axis=1, keepdims=True)
    e0 = jnp.min(jnp.where(masked == m0, iota, EPAD), axis=1, keepdims=True)
    l2 = jnp.where(iota == e0, -1e30, masked)
    m1 = jnp.max(l2, axis=1, keepdims=True)
    e1 = jnp.min(jnp.where(l2 == m1, iota, EPAD), axis=1, keepdims=True)
    w0 = jax.nn.sigmoid(m0 - m1)
    e0_ref[...] = jnp.broadcast_to(e0, (rows, EPAD))
    e1_ref[...] = jnp.broadcast_to(e1, (rows, EPAD))
    w0_ref[...] = jnp.broadcast_to(w0, (rows, EPAD))


def _router(x, gwp):
    rb = 512
    return pl.pallas_call(
        _router_body,
        grid=(T // rb,),
        in_specs=[pl.BlockSpec((rb, H), lambda i: (i, 0)),
                  pl.BlockSpec((EPAD, H), lambda i: (0, 0))],
        out_specs=[pl.BlockSpec((rb, EPAD), lambda i: (i, 0)),
                   pl.BlockSpec((rb, EPAD), lambda i: (i, 0)),
                   pl.BlockSpec((rb, EPAD), lambda i: (i, 0))],
        out_shape=[jax.ShapeDtypeStruct((T, EPAD), jnp.int32),
                   jax.ShapeDtypeStruct((T, EPAD), jnp.int32),
                   jax.ShapeDtypeStruct((T, EPAD), jnp.float32)],
    )(x, gwp)


def _gather_body(x_hbm, tok_hbm, dst_hbm, wsrc_hbm, out_hbm, wrep_hbm,
                 tok_v, dst_v, rows_v, w_v, sem):
    wid = lax.axis_index("s") * NC + lax.axis_index("c")
    pltpu.sync_copy(tok_hbm.at[wid], tok_v)
    pltpu.sync_copy(dst_hbm.at[wid], dst_v)
    cg = pltpu.async_copy(x_hbm.at[tok_v], rows_v, sem)
    pltpu.sync_copy(wsrc_hbm.at[wid], w_v)
    cg.wait()
    cs = pltpu.async_copy(rows_v, out_hbm.at[dst_v], sem)
    cw = pltpu.async_copy(w_v, wrep_hbm.at[dst_v], sem)
    cs.wait()
    cw.wait()


def _gather(xb3, tok2d, dst2d, wsrc):
    mesh = plsc.VectorSubcoreMesh(core_axis_name="c", subcore_axis_name="s",
                                  num_cores=NC)
    return pl.kernel(
        _gather_body,
        out_type=[jax.ShapeDtypeStruct((PADDED, HS, 128), jnp.int32),
                  jax.ShapeDtypeStruct((PADDED, 128), jnp.float32)],
        mesh=mesh,
        scratch_types=[pltpu.VMEM((G_ROWS,), jnp.int32),
                       pltpu.VMEM((G_ROWS,), jnp.int32),
                       pltpu.VMEM((G_ROWS, HS, 128), jnp.int32),
                       pltpu.VMEM((G_ROWS, 128), jnp.float32),
                       pltpu.SemaphoreType.DMA],
    )(xb3, tok2d, dst2d, wsrc)


def _ffn_body(be_ref, xs_ref, wg_ref, wu_ref, wd_ref, wrep_ref, out_ref):
    j = pl.program_id(0)
    i = pl.program_id(1)
    sl = pl.ds(i * M, M)
    xb = xs_ref[sl, :]
    wg = wg_ref[0].astype(jnp.bfloat16)
    wu = wu_ref[0].astype(jnp.bfloat16)
    wd = wd_ref[0].astype(jnp.bfloat16)
    nt = (((1,), (1,)), ((), ()))
    g = lax.dot_general(xb, wg, nt, preferred_element_type=jnp.float32)
    u = lax.dot_general(xb, wu, nt, preferred_element_type=jnp.float32)
    act = (jax.nn.silu(g) * u).astype(jnp.bfloat16)
    y = lax.dot_general(act, wd, nt, preferred_element_type=jnp.float32)

    @pl.when(j == 0)
    def _():
        out_ref[sl, :] = y

    @pl.when(j > 0)
    def _():
        out_ref[sl, :] += y

    @pl.when(j == NJ - 1)
    def _():
        out_ref[sl, :] = out_ref[sl, :] * wrep_ref[sl, 0:1]


def _ffn(block_expert, xs, gate_proj, up_proj, down_proj, w_rep):
    grid_spec = pltpu.PrefetchScalarGridSpec(
        num_scalar_prefetch=1,
        grid=(NJ, NB),
        in_specs=[
            pl.BlockSpec((PADDED, H), lambda j, i, be: (0, 0)),
            pl.BlockSpec((1, FF_BLK, H), lambda j, i, be: (be[i], j, 0)),
            pl.BlockSpec((1, FF_BLK, H), lambda j, i, be: (be[i], j, 0)),
            pl.BlockSpec((1, H, FF_BLK), lambda j, i, be: (be[i], 0, j)),
            pl.BlockSpec((PADDED, 128), lambda j, i, be: (0, 0)),
        ],
        out_specs=pl.BlockSpec((PADDED, H), lambda j, i, be: (0, 0)),
    )
    return pl.pallas_call(
        _ffn_body,
        grid_spec=grid_spec,
        out_shape=jax.ShapeDtypeStruct((PADDED, H), jnp.float32),
        compiler_params=pltpu.CompilerParams(
            dimension_semantics=("arbitrary", "arbitrary")),
    )(block_expert, xs, gate_proj, up_proj, down_proj, w_rep)


def _combine_body(ys_hbm, p0_hbm, p1_hbm, out_hbm, p0_v, p1_v, a_v, b_v, sem):
    wid = lax.axis_index("s") * NC + lax.axis_index("c")
    pltpu.sync_copy(p0_hbm.at[wid], p0_v)
    pltpu.sync_copy(p1_hbm.at[wid], p1_v)
    base = wid * C_NCH * C_CH
    for c in range(C_NCH):
        ca = pltpu.async_copy(ys_hbm.at[p0_v.at[c]], a_v, sem)
        cb = pltpu.async_copy(ys_hbm.at[p1_v.at[c]], b_v, sem)
        ca.wait()
        cb.wait()
        for r in range(C_CH):
            def add_body(t, _, r=r):
                s = pl.ds(t * 16, 16)
                a_v[r, s] = a_v[r, s] + b_v[r, s]
                return 0

            lax.fori_loop(0, H // 16, add_body, 0)
        pltpu.sync_copy(a_v, out_hbm.at[pl.ds(base + c * C_CH, C_CH)])


def _combine(ys, p0_3d, p1_3d):
    mesh = plsc.VectorSubcoreMesh(core_axis_name="c", subcore_axis_name="s",
                                  num_cores=NC)
    return pl.kernel(
        _combine_body,
        out_type=jax.ShapeDtypeStruct((T, H), jnp.float32),
        mesh=mesh,
        scratch_types=[pltpu.VMEM((C_NCH, C_CH), jnp.int32),
                       pltpu.VMEM((C_NCH, C_CH), jnp.int32),
                       pltpu.VMEM((C_CH, H), jnp.float32),
                       pltpu.VMEM((C_CH, H), jnp.float32),
                       pltpu.SemaphoreType.DMA],
    )(ys, p0_3d, p1_3d)


def kernel(hidden_states, gate_w, gate_proj, up_proj, down_proj):
    b, s, h = hidden_states.shape
    assert (b * s, h) == (T, H) and gate_w.shape == (E, H)
    x = hidden_states.reshape(T, H)
    gwp = jnp.zeros((EPAD, H), jnp.float32).at[:E].set(gate_w)

    e0b, e1b, w0b = _router(x, gwp)
    e0, e1, w0 = e0b[:, 0], e1b[:, 0], w0b[:, 0]
    w1 = 1.0 - w0

    flat_e = jnp.stack([e0, e1], axis=1).reshape(-1)
    onehot = (flat_e[:, None] == jnp.arange(E)[None, :]).astype(jnp.int32)
    counts = jnp.sum(onehot, axis=0)
    rank = jnp.sum((jnp.cumsum(onehot, axis=0) - onehot) * onehot, axis=1)
    padded_counts = ((counts + M - 1) // M) * M
    starts = jnp.cumsum(padded_counts) - padded_counts
    dst = (starts[flat_e] + rank).astype(jnp.int32)
    block_expert = (jnp.searchsorted(starts // M, jnp.arange(NB), side="right")
                    - 1).astype(jnp.int32)
    block_expert = jnp.clip(block_expert, 0, E - 1)
    tok = (jnp.arange(T * K, dtype=jnp.int32) // K).reshape(NW, G_ROWS)
    pos = dst.reshape(T, K)

    xb3 = lax.bitcast_convert_type(
        x.astype(jnp.bfloat16).reshape(T, H // 2, 2),
        jnp.int32).reshape(T, HS, 128)
    flat_w = jnp.stack([w0, w1], axis=1).reshape(-1)
    wsrc = jnp.broadcast_to(flat_w[:, None],
                            (T * K, 128)).reshape(NW, G_ROWS, 128)
    xs_i, w_rep = _gather(xb3, tok, dst.reshape(NW, G_ROWS), wsrc)
    xs = lax.bitcast_convert_type(
        xs_i.reshape(PADDED, H // 2), jnp.bfloat16).reshape(PADDED, H)
    ys = _ffn(block_expert, xs, gate_proj, up_proj, down_proj, w_rep)
    out = _combine(ys,
                   pos[:, 0].reshape(NW, C_NCH, C_CH),
                   pos[:, 1].reshape(NW, C_NCH, C_CH))
    return out.reshape(b, s, h)


# speedup vs baseline: 1.3567x; 1.3567x over previous
import jax
import jax.numpy as jnp
from jax import lax
from jax.experimental import pallas as pl
from jax.experimental.pallas import tpu as pltpu
from jax.experimental.pallas import tpu_sc as plsc

T, H, E, K, FF = 2048, 1024, 8, 2, 2816
M = 256
PADDED = T * K + E * M
NB = PADDED // M
FF_BLK = 256
NJ = FF // FF_BLK
EPAD = 128
HS = H // 256

NC, NS = 2, 16
NW = NC * NS
G_ROWS = T * K // NW
C_CH = 32
C_NCH = (T // NW) // C_CH


def _router_body(x_ref, gw_ref, e0_ref, e1_ref, w0_ref):
    logits = lax.dot_general(x_ref[...], gw_ref[...],
                             (((1,), (1,)), ((), ())),
                             preferred_element_type=jnp.float32)
    rows = logits.shape[0]
    iota = lax.broadcasted_iota(jnp.int32, (rows, EPAD), 1)
    masked = jnp.where(iota < E, logits, -1e30)
    m0 = jnp.max(masked, axis=1, keepdims=True)
    e0 = jnp.min(jnp.where(masked == m0, iota, EPAD), axis=1, keepdims=True)
    l2 = jnp.where(iota == e0, -1e30, masked)
    m1 = jnp.max(l2, axis=1, keepdims=True)
    e1 = jnp.min(jnp.where(l2 == m1, iota, EPAD), axis=1, keepdims=True)
    w0 = jax.nn.sigmoid(m0 - m1)
    e0_ref[...] = jnp.broadcast_to(e0, (rows, EPAD))
    e1_ref[...] = jnp.broadcast_to(e1, (rows, EPAD))
    w0_ref[...] = jnp.broadcast_to(w0, (rows, EPAD))


def _router(x, gwp):
    rb = 512
    return pl.pallas_call(
        _router_body,
        grid=(T // rb,),
        in_specs=[pl.BlockSpec((rb, H), lambda i: (i, 0)),
                  pl.BlockSpec((EPAD, H), lambda i: (0, 0))],
        out_specs=[pl.BlockSpec((rb, EPAD), lambda i: (i, 0)),
                   pl.BlockSpec((rb, EPAD), lambda i: (i, 0)),
                   pl.BlockSpec((rb, EPAD), lambda i: (i, 0))],
        out_shape=[jax.ShapeDtypeStruct((T, EPAD), jnp.int32),
                   jax.ShapeDtypeStruct((T, EPAD), jnp.int32),
                   jax.ShapeDtypeStruct((T, EPAD), jnp.float32)],
    )(x, gwp)


def _gather_body(x_hbm, tok_hbm, dst_hbm, wsrc_hbm, out_hbm, wrep_hbm,
                 tok_v, dst_v, rows_v, w_v, sem):
    wid = lax.axis_index("s") * NC + lax.axis_index("c")
    pltpu.sync_copy(tok_hbm.at[wid], tok_v)
    pltpu.sync_copy(dst_hbm.at[wid], dst_v)
    cg = pltpu.async_copy(x_hbm.at[tok_v], rows_v, sem)
    pltpu.sync_copy(wsrc_hbm.at[wid], w_v)
    cg.wait()
    cs = pltpu.async_copy(rows_v, out_hbm.at[dst_v], sem)
    cw = pltpu.async_copy(w_v, wrep_hbm.at[dst_v], sem)
    cs.wait()
    cw.wait()


def _gather(x2, tok2d, dst2d, wsrc):
    mesh = plsc.VectorSubcoreMesh(core_axis_name="c", subcore_axis_name="s",
                                  num_cores=NC)
    return pl.kernel(
        _gather_body,
        out_type=[jax.ShapeDtypeStruct((PADDED, H // 2), jnp.int32),
                  jax.ShapeDtypeStruct((PADDED, 128), jnp.float32)],
        mesh=mesh,
        scratch_types=[pltpu.VMEM((G_ROWS,), jnp.int32),
                       pltpu.VMEM((G_ROWS,), jnp.int32),
                       pltpu.VMEM((G_ROWS, H // 2), jnp.int32),
                       pltpu.VMEM((G_ROWS, 128), jnp.float32),
                       pltpu.SemaphoreType.DMA],
    )(x2, tok2d, dst2d, wsrc)


def _ffn_body(be_ref, xs_ref, wg_ref, wu_ref, wd_ref, wrep_ref, out_ref):
    j = pl.program_id(0)
    i = pl.program_id(1)
    sl = pl.ds(i * M, M)
    xi = xs_ref[sl, :]
    xlo = lax.bitcast_convert_type(xi << 16, jnp.float32).astype(jnp.bfloat16)
    xhi = lax.bitcast_convert_type(xi, jnp.float32).astype(jnp.bfloat16)
    wg = wg_ref[0].astype(jnp.bfloat16)
    wu = wu_ref[0].astype(jnp.bfloat16)
    wd = wd_ref[0].astype(jnp.bfloat16)
    nt = (((1,), (1,)), ((), ()))
    hh = H // 2
    g = (lax.dot_general(xlo, wg[:, :hh], nt, preferred_element_type=jnp.float32)
         + lax.dot_general(xhi, wg[:, hh:], nt, preferred_element_type=jnp.float32))
    u = (lax.dot_general(xlo, wu[:, :hh], nt, preferred_element_type=jnp.float32)
         + lax.dot_general(xhi, wu[:, hh:], nt, preferred_element_type=jnp.float32))
    act = (jax.nn.silu(g) * u).astype(jnp.bfloat16)
    y = lax.dot_general(act, wd, nt, preferred_element_type=jnp.float32)

    @pl.when(j == 0)
    def _():
        out_ref[sl, :] = y

    @pl.when(j > 0)
    def _():
        out_ref[sl, :] += y

    @pl.when(j == NJ - 1)
    def _():
        out_ref[sl, :] = out_ref[sl, :] * wrep_ref[sl, 0:1]


def _ffn(block_expert, xs, gate_proj, up_proj, down_proj, w_rep):
    grid_spec = pltpu.PrefetchScalarGridSpec(
        num_scalar_prefetch=1,
        grid=(NJ, NB),
        in_specs=[
            pl.BlockSpec((PADDED, H // 2), lambda j, i, be: (0, 0)),
            pl.BlockSpec((1, FF_BLK, H), lambda j, i, be: (be[i], j, 0)),
            pl.BlockSpec((1, FF_BLK, H), lambda j, i, be: (be[i], j, 0)),
            pl.BlockSpec((1, H, FF_BLK), lambda j, i, be: (be[i], 0, j)),
            pl.BlockSpec((PADDED, 128), lambda j, i, be: (0, 0)),
        ],
        out_specs=pl.BlockSpec((PADDED, H), lambda j, i, be: (0, 0)),
    )
    return pl.pallas_call(
        _ffn_body,
        grid_spec=grid_spec,
        out_shape=jax.ShapeDtypeStruct((PADDED, H), jnp.float32),
        compiler_params=pltpu.CompilerParams(
            dimension_semantics=("arbitrary", "arbitrary")),
    )(block_expert, xs, gate_proj, up_proj, down_proj, w_rep)


def _combine_body(ys_hbm, p0_hbm, p1_hbm, out_hbm, p0_v, p1_v, a_v, b_v, sem):
    wid = lax.axis_index("s") * NC + lax.axis_index("c")
    pltpu.sync_copy(p0_hbm.at[wid], p0_v)
    pltpu.sync_copy(p1_hbm.at[wid], p1_v)
    base = wid * C_NCH * C_CH
    for c in range(C_NCH):
        ca = pltpu.async_copy(ys_hbm.at[p0_v.at[c]], a_v, sem)
        cb = pltpu.async_copy(ys_hbm.at[p1_v.at[c]], b_v, sem)
        ca.wait()
        cb.wait()
        for r in range(C_CH):
            def add_body(t, _, r=r):
                s = pl.ds(t * 16, 16)
                a_v[r, s] = a_v[r, s] + b_v[r, s]
                return 0

            lax.fori_loop(0, H // 16, add_body, 0)
        pltpu.sync_copy(a_v, out_hbm.at[pl.ds(base + c * C_CH, C_CH)])


def _combine(ys, p0_3d, p1_3d):
    mesh = plsc.VectorSubcoreMesh(core_axis_name="c", subcore_axis_name="s",
                                  num_cores=NC)
    return pl.kernel(
        _combine_body,
        out_type=jax.ShapeDtypeStruct((T, H), jnp.float32),
        mesh=mesh,
        scratch_types=[pltpu.VMEM((C_NCH, C_CH), jnp.int32),
                       pltpu.VMEM((C_NCH, C_CH), jnp.int32),
                       pltpu.VMEM((C_CH, H), jnp.float32),
                       pltpu.VMEM((C_CH, H), jnp.float32),
                       pltpu.SemaphoreType.DMA],
    )(ys, p0_3d, p1_3d)


def kernel(hidden_states, gate_w, gate_proj, up_proj, down_proj):
    b, s, h = hidden_states.shape
    assert (b * s, h) == (T, H) and gate_w.shape == (E, H)
    x = hidden_states.reshape(T, H)
    gwp = jnp.zeros((EPAD, H), jnp.float32).at[:E].set(gate_w)

    e0b, e1b, w0b = _router(x, gwp)
    e0, e1, w0 = e0b[:, 0], e1b[:, 0], w0b[:, 0]
    w1 = 1.0 - w0

    flat_e = jnp.stack([e0, e1], axis=1).reshape(-1)
    onehot = (flat_e[:, None] == jnp.arange(E)[None, :]).astype(jnp.int32)
    counts = jnp.sum(onehot, axis=0)
    rank = jnp.sum((jnp.cumsum(onehot, axis=0) - onehot) * onehot, axis=1)
    padded_counts = ((counts + M - 1) // M) * M
    starts = jnp.cumsum(padded_counts) - padded_counts
    dst = (starts[flat_e] + rank).astype(jnp.int32)
    block_expert = (jnp.searchsorted(starts // M, jnp.arange(NB), side="right")
                    - 1).astype(jnp.int32)
    block_expert = jnp.clip(block_expert, 0, E - 1)
    tok = (jnp.arange(T * K, dtype=jnp.int32) // K).reshape(NW, G_ROWS)
    pos = dst.reshape(T, K)

    xu = lax.bitcast_convert_type(x.astype(jnp.bfloat16), jnp.uint16)
    xu = xu.astype(jnp.uint32)
    x2 = lax.bitcast_convert_type(
        xu[:, :H // 2] | (xu[:, H // 2:] << 16), jnp.int32)
    flat_w = jnp.stack([w0, w1], axis=1).reshape(-1)
    wsrc = jnp.broadcast_to(flat_w[:, None],
                            (T * K, 128)).reshape(NW, G_ROWS, 128)
    xs_i, w_rep = _gather(x2, tok, dst.reshape(NW, G_ROWS), wsrc)
    ys = _ffn(block_expert, xs_i, gate_proj, up_proj, down_proj, w_rep)
    out = _combine(ys,
                   pos[:, 0].reshape(NW, C_NCH, C_CH),
                   pos[:, 1].reshape(NW, C_NCH, C_CH))
    return out.reshape(b, s, h)
